# baseline (speedup 1.0000x reference)
import functools

import jax
import jax.numpy as jnp
from jax import lax
from jax.experimental import pallas as pl
from jax.experimental.pallas import tpu as pltpu
from jax.experimental.pallas import tpu_sc as plsc

_LANES = 16
_QUART = 4096


def _sc_gather_t(t2, idx_flat, nf, vocab, emb_d, bsz):
    info = plsc.get_sparse_core_info()
    nc, ns = info.num_cores, info.num_subcores
    nw = nc * ns
    assert emb_d == nw
    nq = bsz // _QUART
    assert nq >= 2 and bsz % _QUART == 0
    mesh = plsc.VectorSubcoreMesh(core_axis_name="c", subcore_axis_name="s")

    @functools.partial(
        pl.kernel,
        out_type=jax.ShapeDtypeStruct((nf * emb_d * bsz,), jnp.float32),
        mesh=mesh,
        compiler_params=pltpu.CompilerParams(
            use_tc_tiling_on_sc=True, needs_layout_passes=False),
        scratch_types=[
            pltpu.VMEM((vocab,), jnp.float32),
            pltpu.VMEM((bsz,), jnp.int32),
            pltpu.VMEM((_QUART,), jnp.float32),
            pltpu.VMEM((_QUART,), jnp.float32),
            pltpu.SemaphoreType.DMA,
            pltpu.SemaphoreType.DMA,
            pltpu.SemaphoreType.DMA,
            pltpu.SemaphoreType.DMA,
        ],
    )
    def gather_kernel(t2_hbm, idx_hbm, out_hbm, row_v, idx_v, ob0, ob1,
                      rsem, isem, osem0, osem1):
        t = lax.axis_index("s") * nc + lax.axis_index("c")
        obs = (ob0, ob1)
        osems = (osem0, osem1)

        def field_body(i, carry):
            p = i * emb_d + t
            cp_i = pltpu.async_copy(
                idx_hbm.at[pl.ds(i * bsz, bsz)], idx_v, isem)
            cp_r = pltpu.async_copy(t2_hbm.at[p], row_v, rsem)
            cp_i.wait()
            cp_r.wait()
            for q in range(nq):
                ob, osem = obs[q % 2], osems[q % 2]
                dst = out_hbm.at[pl.ds(p * bsz + q * _QUART, _QUART)]

                @pl.when((i > 0) | (q >= 2))
                def _():
                    pltpu.make_async_copy(ob, dst, osem).wait()

                def inner(j, c):
                    b0 = j * _LANES
                    idx16 = idx_v[pl.ds(q * _QUART + b0, _LANES)]
                    ob[pl.ds(b0, _LANES)] = plsc.load_gather(row_v, [idx16])
                    return c

                lax.fori_loop(0, _QUART // _LANES, inner, 0, unroll=16)
                pltpu.async_copy(ob, dst, osem)
            return carry

        lax.fori_loop(0, nf, field_body, 0)
        p_last = (nf - 1) * emb_d + t
        for b in range(2):
            q = nq - 2 + b
            pltpu.make_async_copy(
                obs[q % 2],
                out_hbm.at[pl.ds(p_last * bsz + q * _QUART, _QUART)],
                osems[q % 2]).wait()

    return gather_kernel(t2, idx_flat)


def _mlp_t(emb4, xt, w1et, w1xt, b1c, w2t, b2c, w3t, b3c):
    n_steps = emb4.shape[1]
    d_emb = emb4.shape[0]

    def body(emb_ref, xt_ref, w1et_ref, w1xt_ref, b1_ref, w2t_ref, b2_ref,
             w3t_ref, b3_ref, out_ref):
        e = emb_ref[...].reshape(d_emb, 128).astype(jnp.bfloat16)
        h = jnp.dot(w1et_ref[...], e, preferred_element_type=jnp.float32)
        h = h + jnp.dot(w1xt_ref[...], xt_ref[...].astype(jnp.bfloat16),
                        preferred_element_type=jnp.float32)
        h = jnp.maximum(h + b1_ref[...], 0.0).astype(jnp.bfloat16)
        h = jnp.maximum(
            jnp.dot(w2t_ref[...], h, preferred_element_type=jnp.float32)
            + b2_ref[...], 0.0).astype(jnp.bfloat16)
        out_ref[...] = (
            jnp.dot(w3t_ref[...], h, preferred_element_type=jnp.float32)
            + b3_ref[...]).reshape(1, 1, 128)

    return pl.pallas_call(
        body,
        grid=(n_steps,),
        in_specs=[
            pl.BlockSpec((d_emb, 1, 1, 128), lambda i: (0, i, 0, 0)),
            pl.BlockSpec((xt.shape[0], 128), lambda i: (0, i)),
            pl.BlockSpec(w1et.shape, lambda i: (0, 0)),
            pl.BlockSpec(w1xt.shape, lambda i: (0, 0)),
            pl.BlockSpec(b1c.shape, lambda i: (0, 0)),
            pl.BlockSpec(w2t.shape, lambda i: (0, 0)),
            pl.BlockSpec(b2c.shape, lambda i: (0, 0)),
            pl.BlockSpec(w3t.shape, lambda i: (0, 0)),
            pl.BlockSpec(b3c.shape, lambda i: (0, 0)),
        ],
        out_specs=pl.BlockSpec((1, 1, 128), lambda i: (i, 0, 0)),
        out_shape=jax.ShapeDtypeStruct((n_steps, 1, 128), jnp.float32),
    )(emb4, xt, w1et, w1xt, b1c, w2t, b2c, w3t, b3c)


def kernel(x, x_classes, tables, W1, b1, W2, b2, W3, b3):
    bsz = x.shape[0]
    nf, vocab, emb_d = tables.shape
    d_emb = nf * emb_d
    t2 = tables.transpose(0, 2, 1).reshape(d_emb, vocab)
    idx_flat = x_classes.T.reshape(-1)
    flat = _sc_gather_t(t2, idx_flat, nf, vocab, emb_d, bsz)
    emb4 = flat.reshape(d_emb, bsz // 128, 1, 128)
    xt = x.T
    w1et = W1[:d_emb].T.astype(jnp.bfloat16)
    w1xt = W1[d_emb:].T.astype(jnp.bfloat16)
    outt = _mlp_t(emb4, xt, w1et, w1xt, b1.reshape(-1, 1),
                  W2.T.astype(jnp.bfloat16), b2.reshape(-1, 1),
                  W3.T.astype(jnp.bfloat16), b3.reshape(1, 1))
    return outt.reshape(bsz, 1)


# speedup vs baseline: 1.4741x; 1.4741x over previous
import functools

import jax
import jax.numpy as jnp
from jax import lax
from jax.experimental import pallas as pl
from jax.experimental.pallas import tpu as pltpu
from jax.experimental.pallas import tpu_sc as plsc

_LANES = 16
_QUART = 4096


def _sc_gather_t(t2, idx_flat, nf, vocab, emb_d, bsz):
    info = plsc.get_sparse_core_info()
    nc, ns = info.num_cores, info.num_subcores
    nw = nc * ns
    assert emb_d == nw
    nq = bsz // _QUART
    assert nq >= 2 and bsz % _QUART == 0
    mesh = plsc.VectorSubcoreMesh(core_axis_name="c", subcore_axis_name="s")

    @functools.partial(
        pl.kernel,
        out_type=jax.ShapeDtypeStruct((nf * emb_d * bsz,), jnp.float32),
        mesh=mesh,
        compiler_params=pltpu.CompilerParams(
            use_tc_tiling_on_sc=True, needs_layout_passes=False),
        scratch_types=[
            pltpu.VMEM((vocab,), jnp.float32),
            pltpu.VMEM((bsz,), jnp.int32),
            pltpu.VMEM((_QUART,), jnp.float32),
            pltpu.VMEM((_QUART,), jnp.float32),
            pltpu.SemaphoreType.DMA,
            pltpu.SemaphoreType.DMA,
            pltpu.SemaphoreType.DMA,
            pltpu.SemaphoreType.DMA,
        ],
    )
    def gather_kernel(t2_hbm, idx_hbm, out_hbm, row_v, idx_v, ob0, ob1,
                      rsem, isem, osem0, osem1):
        t = lax.axis_index("s") * nc + lax.axis_index("c")
        obs = (ob0, ob1)
        osems = (osem0, osem1)

        def field_body(i, carry):
            p = i * emb_d + t
            cp_i = pltpu.async_copy(
                idx_hbm.at[pl.ds(i * bsz, bsz)], idx_v, isem)
            cp_r = pltpu.async_copy(t2_hbm.at[p], row_v, rsem)
            cp_i.wait()
            cp_r.wait()
            for q in range(nq):
                ob, osem = obs[q % 2], osems[q % 2]
                dst = out_hbm.at[pl.ds(p * bsz + q * _QUART, _QUART)]

                @pl.when((i > 0) | (q >= 2))
                def _():
                    pltpu.make_async_copy(ob, dst, osem).wait()

                def inner(j, c):
                    b0 = j * (_LANES * 8)
                    idxs = [idx_v[pl.ds(q * _QUART + b0 + k * _LANES, _LANES)]
                            for k in range(8)]
                    vals = [plsc.load_gather(row_v, [ix]) for ix in idxs]
                    for k in range(8):
                        ob[pl.ds(b0 + k * _LANES, _LANES)] = vals[k]
                    return c

                lax.fori_loop(0, _QUART // (_LANES * 8), inner, 0, unroll=2)
                pltpu.async_copy(ob, dst, osem)
            return carry

        lax.fori_loop(0, nf, field_body, 0)
        p_last = (nf - 1) * emb_d + t
        for b in range(2):
            q = nq - 2 + b
            pltpu.make_async_copy(
                obs[q % 2],
                out_hbm.at[pl.ds(p_last * bsz + q * _QUART, _QUART)],
                osems[q % 2]).wait()

    return gather_kernel(t2, idx_flat)


def _mlp_t(emb4, xt, w1et, w1xt, b1c, w2t, b2c, w3t, b3c):
    n_steps = emb4.shape[1]
    d_emb = emb4.shape[0]

    def body(emb_ref, xt_ref, w1et_ref, w1xt_ref, b1_ref, w2t_ref, b2_ref,
             w3t_ref, b3_ref, out_ref):
        e = emb_ref[...].reshape(d_emb, 128).astype(jnp.bfloat16)
        h = jnp.dot(w1et_ref[...], e, preferred_element_type=jnp.float32)
        h = h + jnp.dot(w1xt_ref[...], xt_ref[...].astype(jnp.bfloat16),
                        preferred_element_type=jnp.float32)
        h = jnp.maximum(h + b1_ref[...], 0.0).astype(jnp.bfloat16)
        h = jnp.maximum(
            jnp.dot(w2t_ref[...], h, preferred_element_type=jnp.float32)
            + b2_ref[...], 0.0).astype(jnp.bfloat16)
        out_ref[...] = (
            jnp.dot(w3t_ref[...], h, preferred_element_type=jnp.float32)
            + b3_ref[...]).reshape(1, 1, 128)

    return pl.pallas_call(
        body,
        grid=(n_steps,),
        in_specs=[
            pl.BlockSpec((d_emb, 1, 1, 128), lambda i: (0, i, 0, 0)),
            pl.BlockSpec((xt.shape[0], 128), lambda i: (0, i)),
            pl.BlockSpec(w1et.shape, lambda i: (0, 0)),
            pl.BlockSpec(w1xt.shape, lambda i: (0, 0)),
            pl.BlockSpec(b1c.shape, lambda i: (0, 0)),
            pl.BlockSpec(w2t.shape, lambda i: (0, 0)),
            pl.BlockSpec(b2c.shape, lambda i: (0, 0)),
            pl.BlockSpec(w3t.shape, lambda i: (0, 0)),
            pl.BlockSpec(b3c.shape, lambda i: (0, 0)),
        ],
        out_specs=pl.BlockSpec((1, 1, 128), lambda i: (i, 0, 0)),
        out_shape=jax.ShapeDtypeStruct((n_steps, 1, 128), jnp.float32),
    )(emb4, xt, w1et, w1xt, b1c, w2t, b2c, w3t, b3c)


def kernel(x, x_classes, tables, W1, b1, W2, b2, W3, b3):
    bsz = x.shape[0]
    nf, vocab, emb_d = tables.shape
    d_emb = nf * emb_d
    t2 = tables.transpose(0, 2, 1).reshape(d_emb, vocab)
    idx_flat = x_classes.T.reshape(-1)
    flat = _sc_gather_t(t2, idx_flat, nf, vocab, emb_d, bsz)
    emb4 = flat.reshape(d_emb, bsz // 128, 1, 128)
    xt = x.T
    w1et = W1[:d_emb].T.astype(jnp.bfloat16)
    w1xt = W1[d_emb:].T.astype(jnp.bfloat16)
    outt = _mlp_t(emb4, xt, w1et, w1xt, b1.reshape(-1, 1),
                  W2.T.astype(jnp.bfloat16), b2.reshape(-1, 1),
                  W3.T.astype(jnp.bfloat16), b3.reshape(1, 1))
    return outt.reshape(bsz, 1)


# speedup vs baseline: 1.5419x; 1.0459x over previous
import functools

import jax
import jax.numpy as jnp
from jax import lax
from jax.experimental import pallas as pl
from jax.experimental.pallas import tpu as pltpu
from jax.experimental.pallas import tpu_sc as plsc

_LANES = 16
_QUART = 4096


def _sc_gather_t(t2, idx_flat, nf, vocab, emb_d, bsz):
    info = plsc.get_sparse_core_info()
    nc, ns = info.num_cores, info.num_subcores
    nw = nc * ns
    assert emb_d == nw
    nq = bsz // _QUART
    assert nq >= 2 and bsz % _QUART == 0
    mesh = plsc.VectorSubcoreMesh(core_axis_name="c", subcore_axis_name="s")

    nrow = _QUART // 128

    @functools.partial(
        pl.kernel,
        out_type=jax.ShapeDtypeStruct((bsz // 128, nf * emb_d, 128),
                                      jnp.float32),
        mesh=mesh,
        compiler_params=pltpu.CompilerParams(
            use_tc_tiling_on_sc=True, needs_layout_passes=False),
        scratch_types=[
            pltpu.VMEM((vocab,), jnp.float32),
            pltpu.VMEM((bsz,), jnp.int32),
            pltpu.VMEM((nrow, 128), jnp.float32),
            pltpu.VMEM((nrow, 128), jnp.float32),
            pltpu.SemaphoreType.DMA,
            pltpu.SemaphoreType.DMA,
            pltpu.SemaphoreType.DMA,
            pltpu.SemaphoreType.DMA,
        ],
    )
    def gather_kernel(t2_hbm, idx_hbm, out_hbm, row_v, idx_v, ob0, ob1,
                      rsem, isem, osem0, osem1):
        t = lax.axis_index("s") * nc + lax.axis_index("c")
        obs = (ob0, ob1)
        osems = (osem0, osem1)

        def field_body(i, carry):
            p = i * emb_d + t
            cp_i = pltpu.async_copy(
                idx_hbm.at[pl.ds(i * bsz, bsz)], idx_v, isem)
            cp_r = pltpu.async_copy(t2_hbm.at[p], row_v, rsem)
            cp_i.wait()
            cp_r.wait()
            for q in range(nq):
                ob, osem = obs[q % 2], osems[q % 2]
                dst = out_hbm.at[pl.ds(q * nrow, nrow), p]

                @pl.when((i > 0) | (q >= 2))
                def _():
                    pltpu.make_async_copy(ob, dst, osem).wait()

                def inner(j, c):
                    b0 = q * _QUART + j * 128
                    idxs = [idx_v[pl.ds(b0 + k * _LANES, _LANES)]
                            for k in range(8)]
                    vals = [plsc.load_gather(row_v, [ix]) for ix in idxs]
                    for k in range(8):
                        ob[j, pl.ds(k * _LANES, _LANES)] = vals[k]
                    return c

                lax.fori_loop(0, nrow, inner, 0, unroll=2)
                pltpu.async_copy(ob, dst, osem)
            return carry

        lax.fori_loop(0, nf, field_body, 0)
        p_last = (nf - 1) * emb_d + t
        for b in range(2):
            q = nq - 2 + b
            pltpu.make_async_copy(
                obs[q % 2],
                out_hbm.at[pl.ds(q * nrow, nrow), p_last],
                osems[q % 2]).wait()

    return gather_kernel(t2, idx_flat)


def _mlp_t(emb3, xt, w1et, w1xt, b1c, w2t, b2c, w3t, b3c):
    n_steps = emb3.shape[0]
    d_emb = emb3.shape[1]

    def body(emb_ref, xt_ref, w1et_ref, w1xt_ref, b1_ref, w2t_ref, b2_ref,
             w3t_ref, b3_ref, out_ref):
        e = emb_ref[...].reshape(d_emb, 128)
        h = jnp.dot(w1et_ref[...], e, preferred_element_type=jnp.float32)
        h = h + jnp.dot(w1xt_ref[...], xt_ref[...],
                        preferred_element_type=jnp.float32)
        h = jnp.maximum(h + b1_ref[...], 0.0)
        h = jnp.maximum(
            jnp.dot(w2t_ref[...], h, preferred_element_type=jnp.float32)
            + b2_ref[...], 0.0)
        out_ref[...] = (
            jnp.dot(w3t_ref[...], h, preferred_element_type=jnp.float32)
            + b3_ref[...]).reshape(1, 1, 128)

    return pl.pallas_call(
        body,
        grid=(n_steps,),
        in_specs=[
            pl.BlockSpec((1, d_emb, 128), lambda i: (i, 0, 0)),
            pl.BlockSpec((xt.shape[0], 128), lambda i: (0, i)),
            pl.BlockSpec(w1et.shape, lambda i: (0, 0)),
            pl.BlockSpec(w1xt.shape, lambda i: (0, 0)),
            pl.BlockSpec(b1c.shape, lambda i: (0, 0)),
            pl.BlockSpec(w2t.shape, lambda i: (0, 0)),
            pl.BlockSpec(b2c.shape, lambda i: (0, 0)),
            pl.BlockSpec(w3t.shape, lambda i: (0, 0)),
            pl.BlockSpec(b3c.shape, lambda i: (0, 0)),
        ],
        out_specs=pl.BlockSpec((1, 1, 128), lambda i: (i, 0, 0)),
        out_shape=jax.ShapeDtypeStruct((n_steps, 1, 128), jnp.float32),
    )(emb3, xt, w1et, w1xt, b1c, w2t, b2c, w3t, b3c)


def kernel(x, x_classes, tables, W1, b1, W2, b2, W3, b3):
    bsz = x.shape[0]
    nf, vocab, emb_d = tables.shape
    d_emb = nf * emb_d
    t2 = tables.transpose(0, 2, 1).reshape(d_emb, vocab)
    idx_flat = x_classes.T.reshape(-1)
    emb3 = _sc_gather_t(t2, idx_flat, nf, vocab, emb_d, bsz)
    xt = x.T
    w1et = W1[:d_emb].T
    w1xt = W1[d_emb:].T
    outt = _mlp_t(emb3, xt, w1et, w1xt, b1.reshape(-1, 1),
                  W2.T, b2.reshape(-1, 1), W3.T, b3.reshape(1, 1))
    return outt.reshape(bsz, 1)


# speedup vs baseline: 1.6718x; 1.0843x over previous
import functools

import jax
import jax.numpy as jnp
from jax import lax
from jax.experimental import pallas as pl
from jax.experimental.pallas import tpu as pltpu
from jax.experimental.pallas import tpu_sc as plsc

_LANES = 16
_QUART = 4096


def _sc_gather_t(t2, idx_flat, nf, vocab, emb_d, bsz):
    info = plsc.get_sparse_core_info()
    nc, ns = info.num_cores, info.num_subcores
    nw = nc * ns
    assert emb_d == nw
    nq = bsz // _QUART
    assert nq >= 2 and bsz % _QUART == 0
    mesh = plsc.VectorSubcoreMesh(core_axis_name="c", subcore_axis_name="s")

    nrow = _QUART // 128

    @functools.partial(
        pl.kernel,
        out_type=jax.ShapeDtypeStruct((bsz // 128, nf * emb_d, 128),
                                      jnp.float32),
        mesh=mesh,
        compiler_params=pltpu.CompilerParams(
            use_tc_tiling_on_sc=True, needs_layout_passes=False),
        scratch_types=[
            pltpu.VMEM((vocab,), jnp.float32),
            pltpu.VMEM((bsz,), jnp.int32),
            pltpu.VMEM((nrow, 128), jnp.float32),
            pltpu.VMEM((nrow, 128), jnp.float32),
            pltpu.SemaphoreType.DMA,
            pltpu.SemaphoreType.DMA,
            pltpu.SemaphoreType.DMA,
            pltpu.SemaphoreType.DMA,
        ],
    )
    def gather_kernel(t2_hbm, idx_hbm, out_hbm, row_v, idx_v, ob0, ob1,
                      rsem, isem, osem0, osem1):
        t = lax.axis_index("s") * nc + lax.axis_index("c")
        obs = (ob0, ob1)
        osems = (osem0, osem1)

        def field_body(i, carry):
            p = i * emb_d + t
            cp_i = pltpu.async_copy(
                idx_hbm.at[pl.ds(i * bsz, bsz)], idx_v, isem)
            cp_r = pltpu.async_copy(t2_hbm.at[p], row_v, rsem)
            cp_i.wait()
            cp_r.wait()
            for q in range(nq):
                ob, osem = obs[q % 2], osems[q % 2]
                dst = out_hbm.at[pl.ds(q * nrow, nrow), p]

                @pl.when((i > 0) | (q >= 2))
                def _():
                    pltpu.make_async_copy(ob, dst, osem).wait()

                def inner(j, c):
                    b0 = q * _QUART + j * 128
                    idxs = [idx_v[pl.ds(b0 + k * _LANES, _LANES)]
                            for k in range(8)]
                    vals = [plsc.load_gather(row_v, [ix]) for ix in idxs]
                    for k in range(8):
                        ob[j, pl.ds(k * _LANES, _LANES)] = vals[k]
                    return c

                lax.fori_loop(0, nrow, inner, 0, unroll=2)
                pltpu.async_copy(ob, dst, osem)
            return carry

        lax.fori_loop(0, nf, field_body, 0)
        p_last = (nf - 1) * emb_d + t
        for b in range(2):
            q = nq - 2 + b
            pltpu.make_async_copy(
                obs[q % 2],
                out_hbm.at[pl.ds(q * nrow, nrow), p_last],
                osems[q % 2]).wait()

    return gather_kernel(t2, idx_flat)


_CPS = 4


def _mlp_t(emb3, xt, w1et, w1xt, b1c, w2t, b2c, w3t, b3c):
    n_chunks = emb3.shape[0]
    d_emb = emb3.shape[1]

    def body(emb_ref, xt_ref, w1et_ref, w1xt_ref, b1_ref, w2t_ref, b2_ref,
             w3t_ref, b3_ref, out_ref):
        for k in range(_CPS):
            e = emb_ref[k].astype(jnp.bfloat16)
            h = jnp.dot(w1et_ref[...], e, preferred_element_type=jnp.float32)
            h = h + jnp.dot(w1xt_ref[...], xt_ref[k],
                            preferred_element_type=jnp.float32)
            h = jnp.maximum(h + b1_ref[...], 0.0)
            h = jnp.maximum(
                jnp.dot(w2t_ref[...], h, preferred_element_type=jnp.float32)
                + b2_ref[...], 0.0)
            out_ref[:, k] = (
                jnp.dot(w3t_ref[...], h, preferred_element_type=jnp.float32)
                + b3_ref[...]).reshape(1, 128)

    return pl.pallas_call(
        body,
        grid=(n_chunks // _CPS,),
        in_specs=[
            pl.BlockSpec((_CPS, d_emb, 128), lambda i: (i, 0, 0)),
            pl.BlockSpec((_CPS, xt.shape[1], 128), lambda i: (i, 0, 0)),
            pl.BlockSpec(w1et.shape, lambda i: (0, 0)),
            pl.BlockSpec(w1xt.shape, lambda i: (0, 0)),
            pl.BlockSpec(b1c.shape, lambda i: (0, 0)),
            pl.BlockSpec(w2t.shape, lambda i: (0, 0)),
            pl.BlockSpec(b2c.shape, lambda i: (0, 0)),
            pl.BlockSpec(w3t.shape, lambda i: (0, 0)),
            pl.BlockSpec(b3c.shape, lambda i: (0, 0)),
        ],
        out_specs=pl.BlockSpec((1, _CPS, 128), lambda i: (i, 0, 0)),
        out_shape=jax.ShapeDtypeStruct((n_chunks // _CPS, _CPS, 128),
                                       jnp.float32),
    )(emb3, xt, w1et, w1xt, b1c, w2t, b2c, w3t, b3c)


def kernel(x, x_classes, tables, W1, b1, W2, b2, W3, b3):
    bsz = x.shape[0]
    nf, vocab, emb_d = tables.shape
    d_emb = nf * emb_d
    t2 = tables.transpose(0, 2, 1).reshape(d_emb, vocab)
    idx_flat = x_classes.T.reshape(-1)
    emb3 = _sc_gather_t(t2, idx_flat, nf, vocab, emb_d, bsz)
    xt = x.T.reshape(x.shape[1], bsz // 128, 128).transpose(1, 0, 2)
    w1et = W1[:d_emb].T.astype(jnp.bfloat16)
    w1xt = W1[d_emb:].T
    outt = _mlp_t(emb3, xt, w1et, w1xt, b1.reshape(-1, 1),
                  W2.T, b2.reshape(-1, 1), W3.T, b3.reshape(1, 1))
    return outt.reshape(bsz, 1)
